# Initial kernel scaffold; baseline (speedup 1.0000x reference)
import jax
import jax.numpy as jnp
from jax.experimental import pallas as pl


def kernel(input_args, embed_tokens_weight):
    raise NotImplementedError("write your pallas kernel here")


# speedup vs baseline: 1.8261x; 1.8261x over previous
import functools

import jax
import jax.numpy as jnp
from jax import lax
from jax.experimental import pallas as pl
from jax.experimental.pallas import tpu as pltpu
from jax.experimental.pallas import tpu_sc as plsc

VOCAB = 32000
HIDDEN = 4096
BATCH = 4
SEQ = 4096
NTOK = BATCH * SEQ
NC = 2
NS = 16
NW = NC * NS
PER_W = NTOK // NW
CHUNK = 8
NCHUNK = PER_W // CHUNK
NBUF = 2

_mesh = plsc.VectorSubcoreMesh(core_axis_name="c", subcore_axis_name="s")


@functools.partial(
    pl.kernel,
    out_type=jax.ShapeDtypeStruct((NTOK, HIDDEN), jnp.float32),
    mesh=_mesh,
    scratch_types=[
        pltpu.VMEM((NCHUNK, CHUNK), jnp.int32),
        pltpu.VMEM((CHUNK, HIDDEN), jnp.float32),
        pltpu.VMEM((CHUNK, HIDDEN), jnp.float32),
        pltpu.SemaphoreType.DMA,
        pltpu.SemaphoreType.DMA,
        pltpu.SemaphoreType.DMA,
        pltpu.SemaphoreType.DMA,
    ],
)
def _emb_lookup(idx_hbm, table_hbm, out_hbm, idx_v, buf0, buf1, g0, g1, s0, s1):
    wid = lax.axis_index("s") * NC + lax.axis_index("c")
    base = wid * PER_W
    bufs = (buf0, buf1)
    gsems = (g0, g1)
    ssems = (s0, s1)

    pltpu.sync_copy(idx_hbm.at[wid], idx_v)

    def gather_start(c, b):
        pltpu.async_copy(table_hbm.at[idx_v.at[c]], bufs[b], gsems[b])

    def gather_wait(c, b):
        pltpu.make_async_copy(table_hbm.at[idx_v.at[c]], bufs[b], gsems[b]).wait()

    def store_start(c, b):
        pltpu.async_copy(
            bufs[b], out_hbm.at[pl.ds(base + c * CHUNK, CHUNK)], ssems[b])

    def store_wait(c, b):
        pltpu.make_async_copy(
            bufs[b], out_hbm.at[pl.ds(base + c * CHUNK, CHUNK)], ssems[b]).wait()

    for b in range(NBUF):
        gather_start(b, b)

    def step(i, carry):
        for b in range(NBUF):
            c = NBUF * i + b
            gather_wait(c, b)
            store_start(c, b)
            store_wait(c, b)
            gather_start(c + NBUF, b)
        return carry

    lax.fori_loop(0, NCHUNK // NBUF - 1, step, 0)

    for b in range(NBUF):
        c = NCHUNK - NBUF + b
        gather_wait(c, b)
        store_start(c, b)
        store_wait(c, b)


def kernel(input_args, embed_tokens_weight):
    idx = input_args.reshape(NW, NCHUNK, CHUNK).astype(jnp.int32)
    out = _emb_lookup(idx, embed_tokens_weight)
    return out.reshape(BATCH, SEQ, HIDDEN)
